# Initial kernel scaffold; baseline (speedup 1.0000x reference)
import jax
import jax.numpy as jnp
from jax.experimental import pallas as pl


def kernel(input_ids, token_type_ids, positions, hidden_state_norms, layer_idx, token_type_table):
    raise NotImplementedError("write your pallas kernel here")


# speedup vs baseline: 1.0412x; 1.0412x over previous
import functools

import jax
import jax.numpy as jnp
from jax import lax
from jax.experimental import pallas as pl
from jax.experimental.pallas import tpu as pltpu
from jax.experimental.pallas import tpu_sc as plsc

B, S = 4, 8192
F = 1024
Q = 256
N = B * S
NC, NS = 2, 16
NW = NC * NS
RPW = N // NW
CH = 64
NCH = RPW // CH
SLABW = 16
ZW = F - Q - SLABW
L = 16


def _body(table, types3, pos_all, ids_all, hsn_all, lay,
          out,
          emb0, emb1, slab0, slab1, zbuf, hsnbuf, types_v, pos_v, ids_v, lay_v,
          sg0, sg1, so0, so1):
    wid = lax.axis_index("s") * NC + lax.axis_index("c")
    row0w = wid * RPW
    iota = lax.iota(jnp.int32, L)
    z16 = jnp.zeros((L,), jnp.float32)

    pltpu.sync_copy(hsn_all, hsnbuf)

    def _mx(i, acc):
        return jnp.maximum(acc, hsnbuf[pl.ds(i * L, L)])

    maxv = jnp.max(lax.fori_loop(0, N // L, _mx, jnp.full((L,), -1e30, jnp.float32)))

    def _zrow(r, carry):
        for k in range(ZW // L):
            zbuf[r, pl.ds(k * L, L)] = z16
        return carry

    lax.fori_loop(0, CH, _zrow, 0)

    pltpu.sync_copy(lay, lay_v)
    layv = lay_v[...]
    for slab in (slab0, slab1):
        for g in range(CH // L):
            rows = g * L + iota
            plsc.store_scatter(slab, [rows, jnp.full((L,), 3, jnp.int32)], layv)
            for col in range(4, SLABW):
                plsc.store_scatter(slab, [rows, jnp.full((L,), col, jnp.int32)], z16)

    pltpu.sync_copy(types3.at[wid], types_v)
    pltpu.sync_copy(pos_all.at[pl.ds(row0w, RPW)], pos_v)
    pltpu.sync_copy(ids_all.at[pl.ds(row0w, RPW)], ids_v)

    embs = (emb0, emb1)
    slabs = (slab0, slab1)
    sgs = (sg0, sg1)
    sos = (so0, so1)
    outstanding = [None, None]

    for c in range(NCH):
        b = c % 2
        emb, slab, sg, so = embs[b], slabs[b], sgs[b], sos[b]
        if outstanding[b] is not None:
            for h in outstanding[b]:
                h.wait()
        gh = pltpu.async_copy(table.at[types_v.at[c]], emb, sg)
        for g in range(CH // L):
            off = c * CH + g * L
            rows = g * L + iota
            posv = pos_v[pl.ds(off, L)].astype(jnp.float32) * (1.0 / S)
            specv = jnp.where(ids_v[pl.ds(off, L)] < 5, 1.0, 0.0).astype(jnp.float32)
            hv = hsnbuf[pl.ds(row0w + off, L)] / maxv
            plsc.store_scatter(slab, [rows, jnp.full((L,), 0, jnp.int32)], posv)
            plsc.store_scatter(slab, [rows, jnp.full((L,), 1, jnp.int32)], specv)
            plsc.store_scatter(slab, [rows, jnp.full((L,), 2, jnp.int32)], hv)
        gh.wait()
        row0 = row0w + c * CH
        h1 = pltpu.async_copy(emb, out.at[pl.ds(row0, CH), pl.ds(0, Q)], so)
        h2 = pltpu.async_copy(slab, out.at[pl.ds(row0, CH), pl.ds(Q, SLABW)], so)
        h3 = pltpu.async_copy(zbuf, out.at[pl.ds(row0, CH), pl.ds(Q + SLABW, ZW)], so)
        outstanding[b] = [h1, h2, h3]

    for b in range(2):
        for h in outstanding[b]:
            h.wait()


@functools.partial(jax.jit, static_argnames=())
def _run(table, types3, pos_all, ids_all, hsn_all, lay):
    mesh = plsc.VectorSubcoreMesh(
        core_axis_name="c", subcore_axis_name="s", num_cores=NC, num_subcores=NS
    )
    f = functools.partial(
        pl.kernel,
        out_type=jax.ShapeDtypeStruct((N, F), jnp.float32),
        mesh=mesh,
        scratch_types=[
            pltpu.VMEM((CH, Q), jnp.float32),
            pltpu.VMEM((CH, Q), jnp.float32),
            pltpu.VMEM((CH, SLABW), jnp.float32),
            pltpu.VMEM((CH, SLABW), jnp.float32),
            pltpu.VMEM((CH, ZW), jnp.float32),
            pltpu.VMEM((N,), jnp.float32),
            pltpu.VMEM((NCH, CH), jnp.int32),
            pltpu.VMEM((RPW,), jnp.int32),
            pltpu.VMEM((RPW,), jnp.int32),
            pltpu.VMEM((L,), jnp.float32),
            pltpu.SemaphoreType.DMA,
            pltpu.SemaphoreType.DMA,
            pltpu.SemaphoreType.DMA,
            pltpu.SemaphoreType.DMA,
        ],
        compiler_params=pltpu.CompilerParams(
            use_tc_tiling_on_sc=False, needs_layout_passes=False
        ),
    )(_body)
    return f(table, types3, pos_all, ids_all, hsn_all, lay)


def kernel(input_ids, token_type_ids, positions, hidden_state_norms,
           layer_idx, token_type_table):
    types3 = token_type_ids.reshape(NW, NCH, CH)
    pos_all = positions.reshape(N)
    ids_all = input_ids.reshape(N)
    hsn_all = hidden_state_norms.reshape(N)
    lay = jnp.zeros((L,), jnp.float32) + jnp.asarray(layer_idx, jnp.float32) / 100.0
    out = _run(token_type_table, types3, pos_all, ids_all, hsn_all, lay)
    return out.reshape(B, S, F)
